# baseline (speedup 1.0000x reference)
import math

import jax
import jax.numpy as jnp
from jax import lax
from jax.experimental import pallas as pl
from jax.experimental.pallas import tpu as pltpu

N = 8
D = 128
T = 8
NF = 6
VR = 16.0
NS = 3
BB = 512
NP = 4

_PAIRS = [
    [(i, i ^ (1 << s)) for i in range(N) if i < (i ^ (1 << s))]
    for s in range(NS)
]
_SQRT2 = math.sqrt(2.0)


def _dot(a, b):
    return jnp.dot(a, b, preferred_element_type=jnp.float32)


def _gelu(v):
    return 0.5 * v * (1.0 + lax.erf(v / _SQRT2))


def _gelu_pre(z):
    t = 0.5 * z
    return t * lax.erf(z) + t


def _first_argmax(l):
    m = jnp.max(l, axis=1, keepdims=True)
    iota = lax.broadcasted_iota(jnp.int32, l.shape, 1)
    return jnp.min(jnp.where(l >= m, iota, T), axis=1, keepdims=True)


def _body(x_ref, ve_w_ref, ve_b_ref, ln_w_ref, ln_b_ref,
          rww_ref, rb1_ref, rw2d_ref, rb2d_ref,
          tw1_ref, tb1_ref, tw2_ref, tb2_ref,
          hw1_ref, hb1_ref, hw2_ref, hb2_ref, out_ref):
    bb = x_ref.shape[0]
    freqs = (2.0 ** lax.broadcasted_iota(jnp.int32, (1, NF), 1).astype(
        jnp.float32)) * (2.0 * math.pi / VR)

    xall = jnp.concatenate([x_ref[:, i:i + 1] for i in range(N)], axis=0)
    ang = xall * freqs
    feat = jnp.concatenate([jnp.sin(ang), jnp.cos(ang)], axis=1)
    emb = _dot(feat, ve_w_ref[:]) + ve_b_ref[:]
    mu = jnp.mean(emb, axis=1, keepdims=True)
    var = jnp.mean((emb - mu) ** 2, axis=1, keepdims=True)
    emb = (emb - mu) * lax.rsqrt(var + 1e-5) * ln_w_ref[:] + ln_b_ref[:]
    vals = [emb[i * bb:(i + 1) * bb] for i in range(N)]

    for s in range(NS):
        pair = jnp.concatenate(
            [jnp.concatenate([vals[i], vals[j]], axis=1)
             for (i, j) in _PAIRS[s]], axis=0)
        b1 = jnp.concatenate(
            [jnp.broadcast_to(rb1_ref[s, p], (bb, 2 * D))
             for p in range(NP)], axis=0)
        h12 = _gelu(_dot(pair, rww_ref[s]) + b1)
        l12 = _dot(h12, rw2d_ref[s]) + rb2d_ref[s]
        idx1 = _first_argmax(l12[:, :T])
        idx2 = _first_argmax(l12[:, T:])
        out1 = jnp.zeros((NP * bb, D), jnp.float32)
        out2 = jnp.zeros((NP * bb, D), jnp.float32)
        for t in range(T):
            u = _gelu(_dot(pair, tw1_ref[s, t]) + tb1_ref[s, t])
            o = _dot(u, tw2_ref[s, t]) + tb2_ref[s, t]
            out1 = out1 + jnp.where(idx1 == t, o, 0.0)
            out2 = out2 + jnp.where(idx2 == t, o, 0.0)
        for p, (i, j) in enumerate(_PAIRS[s]):
            vals[i] = out1[p * bb:(p + 1) * bb]
            vals[j] = out2[p * bb:(p + 1) * bb]

    v = jnp.concatenate(vals, axis=0)
    h = _gelu(_dot(v, hw1_ref[:]) + hb1_ref[:])
    o = _dot(h, hw2_ref[:]) + hb2_ref[:]
    out_ref[:] = jnp.concatenate(
        [o[i * bb:(i + 1) * bb] for i in range(N)], axis=1)


def kernel(x, params):
    bs = x.shape[0]
    st = params["stages"]

    rw = jnp.stack([st[s]["r_w1"][:2 * D] for s in range(NS)])
    rwsw = jnp.stack([
        jnp.concatenate([st[s]["r_w1"][D:2 * D], st[s]["r_w1"][:D]], axis=0)
        for s in range(NS)
    ])
    rww = jnp.concatenate([rw, rwsw], axis=2)
    rb1 = jnp.stack([
        jnp.stack([
            st[s]["r_b1"]
            + jnp.concatenate([params["pos_embed"][i],
                               params["stage_embed"][s]]) @ st[s]["r_w1"][2 * D:]
            for (i, _) in _PAIRS[s]
        ])
        for s in range(NS)
    ])
    rb1 = jnp.concatenate([rb1, rb1], axis=2).reshape(NS, NP, 1, 2 * D)
    rw2 = jnp.stack([st[s]["r_w2"] for s in range(NS)])
    z = jnp.zeros_like(rw2)
    rw2d = jnp.concatenate(
        [jnp.concatenate([rw2, z], axis=2),
         jnp.concatenate([z, rw2], axis=2)], axis=1)
    rb2 = jnp.stack([st[s]["r_b2"] for s in range(NS)])
    rb2d = jnp.concatenate([rb2, rb2], axis=1).reshape(NS, 1, 2 * T)
    tw1 = jnp.stack([st[s]["t_w1"] for s in range(NS)])
    tb1 = jnp.stack([st[s]["t_b1"] for s in range(NS)]).reshape(NS, T, 1, 2 * D)
    tw2 = jnp.stack([st[s]["t_w2"] for s in range(NS)])
    tb2 = jnp.stack([st[s]["t_b2"] for s in range(NS)]).reshape(NS, T, 1, D)

    ve_b = params["ve_b"].reshape(1, D)
    ln_w = params["ln_w"].reshape(1, D)
    ln_b = params["ln_b"].reshape(1, D)
    hb1 = params["head_b1"].reshape(1, D)
    hb2 = params["head_b2"].reshape(1, 1)

    weights = (params["ve_w"], ve_b, ln_w, ln_b,
               rww, rb1, rw2d, rb2d,
               tw1, tb1, tw2, tb2,
               params["head_w1"], hb1, params["head_w2"], hb2)

    def full(a):
        return pl.BlockSpec(a.shape, lambda b, _n=a.ndim: (0,) * _n)

    out = pl.pallas_call(
        _body,
        grid=(bs // BB,),
        in_specs=[pl.BlockSpec((BB, N), lambda b: (b, 0))]
        + [full(w) for w in weights],
        out_specs=pl.BlockSpec((BB, N), lambda b: (b, 0)),
        out_shape=jax.ShapeDtypeStruct((bs, N), jnp.float32),
        compiler_params=pltpu.CompilerParams(
            dimension_semantics=("arbitrary",),
        ),
    )(x, *weights)
    return out


# speedup vs baseline: 1.0576x; 1.0576x over previous
import math

import jax
import jax.numpy as jnp
from jax import lax
from jax.experimental import pallas as pl
from jax.experimental.pallas import tpu as pltpu

N = 8
D = 128
T = 8
NF = 6
VR = 16.0
NS = 3
BB = 2048
NP = 4

_PAIRS = [
    [(i, i ^ (1 << s)) for i in range(N) if i < (i ^ (1 << s))]
    for s in range(NS)
]
_SQRT2 = math.sqrt(2.0)


def _dot(a, b):
    return jnp.dot(a, b, preferred_element_type=jnp.float32)


def _gelu(v):
    return 0.5 * v * (1.0 + lax.erf(v / _SQRT2))


def _gelu_pre(z):
    t = 0.5 * z
    return t * lax.erf(z) + t


def _first_argmax(l):
    m = jnp.max(l, axis=1, keepdims=True)
    iota = lax.broadcasted_iota(jnp.int32, l.shape, 1)
    return jnp.min(jnp.where(l >= m, iota, T), axis=1, keepdims=True)


def _body(x_ref, ve_w_ref, ve_b_ref, ln_w_ref, ln_b_ref,
          rww_ref, rb1_ref, rw2d_ref, rb2d_ref,
          tw1_ref, tb1_ref, tw2_ref, tb2_ref,
          hw1_ref, hb1_ref, hw2_ref, hb2_ref, out_ref):
    bb = x_ref.shape[0]
    freqs = (2.0 ** lax.broadcasted_iota(jnp.int32, (1, NF), 1).astype(
        jnp.float32)) * (2.0 * math.pi / VR)

    xall = jnp.concatenate([x_ref[:, i:i + 1] for i in range(N)], axis=0)
    ang = xall * freqs
    feat = jnp.concatenate([jnp.sin(ang), jnp.cos(ang)], axis=1)
    emb = _dot(feat, ve_w_ref[:]) + ve_b_ref[:]
    mu = jnp.mean(emb, axis=1, keepdims=True)
    var = jnp.mean((emb - mu) ** 2, axis=1, keepdims=True)
    emb = (emb - mu) * lax.rsqrt(var + 1e-5) * ln_w_ref[:] + ln_b_ref[:]
    vals = [emb[i * bb:(i + 1) * bb] for i in range(N)]

    for s in range(NS):
        pair = jnp.concatenate(
            [jnp.concatenate([vals[i], vals[j]], axis=1)
             for (i, j) in _PAIRS[s]], axis=0)
        b1 = jnp.concatenate(
            [jnp.broadcast_to(rb1_ref[s, p], (bb, 2 * D))
             for p in range(NP)], axis=0)
        h12 = _gelu(_dot(pair, rww_ref[s]) + b1)
        l12 = _dot(h12, rw2d_ref[s]) + rb2d_ref[s]
        idx1 = _first_argmax(l12[:, :T])
        idx2 = _first_argmax(l12[:, T:])
        out1 = jnp.zeros((NP * bb, D), jnp.float32)
        out2 = jnp.zeros((NP * bb, D), jnp.float32)
        for t in range(T):
            u = _gelu(_dot(pair, tw1_ref[s, t]) + tb1_ref[s, t])
            o = _dot(u, tw2_ref[s, t]) + tb2_ref[s, t]
            out1 = out1 + jnp.where(idx1 == t, o, 0.0)
            out2 = out2 + jnp.where(idx2 == t, o, 0.0)
        for p, (i, j) in enumerate(_PAIRS[s]):
            vals[i] = out1[p * bb:(p + 1) * bb]
            vals[j] = out2[p * bb:(p + 1) * bb]

    v = jnp.concatenate(vals, axis=0)
    h = _gelu(_dot(v, hw1_ref[:]) + hb1_ref[:])
    o = _dot(h, hw2_ref[:]) + hb2_ref[:]
    out_ref[:] = jnp.concatenate(
        [o[i * bb:(i + 1) * bb] for i in range(N)], axis=1)


def kernel(x, params):
    bs = x.shape[0]
    st = params["stages"]

    rw = jnp.stack([st[s]["r_w1"][:2 * D] for s in range(NS)])
    rwsw = jnp.stack([
        jnp.concatenate([st[s]["r_w1"][D:2 * D], st[s]["r_w1"][:D]], axis=0)
        for s in range(NS)
    ])
    rww = jnp.concatenate([rw, rwsw], axis=2)
    rb1 = jnp.stack([
        jnp.stack([
            st[s]["r_b1"]
            + jnp.concatenate([params["pos_embed"][i],
                               params["stage_embed"][s]]) @ st[s]["r_w1"][2 * D:]
            for (i, _) in _PAIRS[s]
        ])
        for s in range(NS)
    ])
    rb1 = jnp.concatenate([rb1, rb1], axis=2).reshape(NS, NP, 1, 2 * D)
    rw2 = jnp.stack([st[s]["r_w2"] for s in range(NS)])
    z = jnp.zeros_like(rw2)
    rw2d = jnp.concatenate(
        [jnp.concatenate([rw2, z], axis=2),
         jnp.concatenate([z, rw2], axis=2)], axis=1)
    rb2 = jnp.stack([st[s]["r_b2"] for s in range(NS)])
    rb2d = jnp.concatenate([rb2, rb2], axis=1).reshape(NS, 1, 2 * T)
    tw1 = jnp.stack([st[s]["t_w1"] for s in range(NS)])
    tb1 = jnp.stack([st[s]["t_b1"] for s in range(NS)]).reshape(NS, T, 1, 2 * D)
    tw2 = jnp.stack([st[s]["t_w2"] for s in range(NS)])
    tb2 = jnp.stack([st[s]["t_b2"] for s in range(NS)]).reshape(NS, T, 1, D)

    ve_b = params["ve_b"].reshape(1, D)
    ln_w = params["ln_w"].reshape(1, D)
    ln_b = params["ln_b"].reshape(1, D)
    hb1 = params["head_b1"].reshape(1, D)
    hb2 = params["head_b2"].reshape(1, 1)

    weights = (params["ve_w"], ve_b, ln_w, ln_b,
               rww, rb1, rw2d, rb2d,
               tw1, tb1, tw2, tb2,
               params["head_w1"], hb1, params["head_w2"], hb2)

    def full(a):
        return pl.BlockSpec(a.shape, lambda b, _n=a.ndim: (0,) * _n)

    out = pl.pallas_call(
        _body,
        grid=(bs // BB,),
        in_specs=[pl.BlockSpec((BB, N), lambda b: (b, 0))]
        + [full(w) for w in weights],
        out_specs=pl.BlockSpec((BB, N), lambda b: (b, 0)),
        out_shape=jax.ShapeDtypeStruct((bs, N), jnp.float32),
        compiler_params=pltpu.CompilerParams(
            dimension_semantics=("arbitrary",),
        ),
    )(x, *weights)
    return out
